# Initial kernel scaffold; baseline (speedup 1.0000x reference)
import jax
import jax.numpy as jnp
from jax.experimental import pallas as pl


def kernel(x, edge_index, batch, W1, b1, W2, b2, W3, b3, W4, b4, cw1, cb1, cw2, cb2, fc1_W, fc1_b, fc3_W, fc3_b):
    raise NotImplementedError("write your pallas kernel here")


# speedup vs baseline: 12.2916x; 12.2916x over previous
import functools
import jax
import jax.numpy as jnp
import numpy as np
from jax import lax
from jax.experimental import pallas as pl
from jax.experimental.pallas import tpu as pltpu
from jax.experimental.pallas import tpu_sc as plsc

N_NODES = 100000
N_EDGES = 1600000
N_GRAPHS = 64
KTOP = 100
NS = 16
NCORES = 2
CHUNK = 1024
NROW = CHUNK // 128
EPT = N_EDGES // NS
NCHUNK = -(-EPT // CHUNK)
PEPT = NCHUNK * CHUNK
HALF = 16
TROWS = 2 * N_NODES + 8


def _agg_body(tab_hbm, src_hbm, dst_hbm, zeros_hbm, out_hbm,
              src_v, dst_v, rows_v, acc, sem):
    c = lax.axis_index("c")
    s = lax.axis_index("s")

    @pl.when(s == 0)
    def _zero():
        pltpu.sync_copy(zeros_hbm, acc)

    plsc.subcore_barrier()

    def body(j, carry):
        srow = ((c * NS + s) * NCHUNK + j) * NROW
        drow = (s * NCHUNK + j) * NROW
        pltpu.sync_copy(src_hbm.at[pl.ds(srow, NROW)], src_v)
        pltpu.sync_copy(dst_hbm.at[pl.ds(drow, NROW)], dst_v)
        for k in range(NROW):
            pltpu.async_copy(tab_hbm.at[src_v.at[k]],
                             rows_v.at[pl.ds(k * 128, 128)], sem).wait()
            pltpu.sync_copy(rows_v.at[pl.ds(k * 128, 128)],
                            acc.at[dst_v.at[k]], add=True)
        return carry

    lax.fori_loop(0, NCHUNK, body, 0)
    plsc.subcore_barrier()

    @pl.when(s == 0)
    def _drain():
        pltpu.sync_copy(acc, out_hbm.at[pl.ds(c * N_NODES, N_NODES)])


@functools.cache
def _make_agg_call():
    return functools.partial(
        pl.kernel,
        mesh=plsc.VectorSubcoreMesh(core_axis_name="c", subcore_axis_name="s"),
        compiler_params=pltpu.CompilerParams(use_tc_tiling_on_sc=False),
        out_type=jax.ShapeDtypeStruct((2 * N_NODES, HALF), jnp.float32),
        scratch_types=[
            pltpu.VMEM((NROW, 128), jnp.int32),
            pltpu.VMEM((NROW, 128), jnp.int32),
            pltpu.VMEM((CHUNK, HALF), jnp.float32),
            pltpu.VMEM_SHARED((N_NODES, HALF), jnp.float32),
            pltpu.SemaphoreType.DMA,
        ],
    )(_agg_body)


def _sc_aggregate(g, src2, dstp, zeros):
    tab = jnp.concatenate(
        [g[:, :HALF], g[:, HALF:], jnp.zeros((8, HALF), jnp.float32)], axis=0)
    out = _make_agg_call()(tab, src2, dstp, zeros)
    return jnp.concatenate([out[:N_NODES], out[N_NODES:]], axis=1)


def _global_sort_pool(h, batch, k, num_graphs):
    n, d = h.shape
    counts = jnp.bincount(batch, length=num_graphs)
    starts = jnp.concatenate([jnp.zeros((1,), counts.dtype),
                              jnp.cumsum(counts)[:-1]])
    perm = jnp.lexsort((-h[:, -1], batch))
    hs = h[perm]
    idx = starts[:, None] + jnp.arange(k)[None, :]
    valid = jnp.arange(k)[None, :] < counts[:, None]
    gathered = hs[jnp.clip(idx, 0, n - 1)]
    pooled = jnp.where(valid[:, :, None], gathered, 0.0)
    return pooled.reshape(num_graphs, k * d)


def kernel(x, edge_index, batch, W1, b1, W2, b2, W3, b3, W4, b4,
           cw1, cb1, cw2, cb2, fc1_W, fc1_b, fc3_W, fc3_b):
    src = edge_index[0].astype(jnp.int32)
    dst = edge_index[1].astype(jnp.int32)

    npad = PEPT - EPT
    srcr = src.reshape(NS, EPT)
    padi = jnp.full((NS, npad), 2 * N_NODES, jnp.int32)
    s0 = jnp.concatenate([srcr, padi], axis=1)
    s1 = jnp.concatenate([srcr + N_NODES, padi], axis=1)
    src2 = jnp.stack([s0, s1]).reshape(-1, 128)
    dstp = jnp.concatenate(
        [dst.reshape(NS, EPT), jnp.zeros((NS, npad), jnp.int32)],
        axis=1).reshape(-1, 128)
    zeros = jnp.zeros((N_NODES, HALF), jnp.float32)

    ones_tab = jnp.ones((N_NODES, 2 * HALF), jnp.float32)
    indeg = _sc_aggregate(ones_tab, src2, dstp, zeros)[:, 0]
    dinv = lax.rsqrt(indeg + 1.0)

    def gcn_pre(h):
        g = h * dinv[:, None]
        return dinv[:, None] * (_sc_aggregate(g, src2, dstp, zeros) + g)

    x1 = jnp.tanh(gcn_pre(x @ W1) + b1)
    x2 = jnp.tanh(gcn_pre(x1 @ W2) + b2)
    a2 = gcn_pre(x2)
    x3 = jnp.tanh(a2 @ W3 + b3)
    x4 = jnp.tanh(a2 @ W4 + b4)

    h = jnp.concatenate([x1, x2, x3, x4], axis=1)
    p = _global_sort_pool(h, batch, KTOP, N_GRAPHS)

    t = p.reshape(-1, 1, KTOP * h.shape[1])
    c = jax.nn.relu(lax.conv_general_dilated(
        t, cw1, (h.shape[1],), 'VALID',
        dimension_numbers=('NCH', 'OIH', 'NCH')) + cb1[None, :, None])
    c = lax.reduce_window(c, -jnp.inf, lax.max, (1, 1, 2), (1, 1, 2), 'VALID')
    c = jax.nn.relu(lax.conv_general_dilated(
        c, cw2, (1,), 'VALID',
        dimension_numbers=('NCH', 'OIH', 'NCH')) + cb2[None, :, None])
    flat = c.reshape(N_GRAPHS, -1)
    h1 = jax.nn.relu(flat @ fc1_W + fc1_b)
    logits = h1 @ fc3_W + fc3_b
    return jax.nn.log_softmax(logits, axis=-1)
